# baseline (speedup 1.0000x reference)
import functools

import jax
import jax.numpy as jnp
from jax import lax
from jax.experimental import pallas as pl
from jax.experimental.pallas import tpu as pltpu
from jax.experimental.pallas import tpu_sc as plsc

N = 10000
E = 320000
NC = 2
NS = 16
NW = NC * NS
CHUNK = 128
NCHUNK_A = 79
EPT_A = NCHUNK_A * CHUNK
EPAD_A = EPT_A * NW
NBUF = 2
SUB = 2
NG = 80
GSZ = SUB * CHUNK
EPT_B = NG * GSZ
EPAD_B = EPT_B * NS
NPAD = 10112
STRIPE = NPAD // NS
HQ = 32

_mesh = plsc.VectorSubcoreMesh(
    core_axis_name="c", subcore_axis_name="s", num_cores=NC, num_subcores=NS)


@functools.partial(
    pl.kernel,
    out_type=jax.ShapeDtypeStruct((NC, NPAD, 16), jnp.float32),
    mesh=_mesh,
    scratch_types=[
        pltpu.VMEM((NCHUNK_A, CHUNK), jnp.int32),
        pltpu.VMEM((CHUNK, 16), jnp.float32),
        pltpu.VMEM((320, 16), jnp.float32),
        pltpu.VMEM_SHARED((NPAD, 16), jnp.float32),
    ],
    compiler_params=pltpu.CompilerParams(use_tc_tiling_on_sc=False),
)
def _deg_kernel(col_hbm, out_hbm, colv, ones_v, zbuf, acc):
    c = lax.axis_index("c")
    s = lax.axis_index("s")
    wid = c * NS + s
    pltpu.sync_copy(col_hbm.at[wid], colv)

    def fill(i, carry):
        ones_v[i, :] = jnp.ones((16,), jnp.float32)
        return carry

    lax.fori_loop(0, CHUNK, fill, 0)

    def fill_zero(i, carry):
        zbuf[i, :] = jnp.zeros((16,), jnp.float32)
        return carry

    lax.fori_loop(0, 320, fill_zero, 0)

    base = s * STRIPE
    pltpu.sync_copy(zbuf, acc.at[pl.ds(base, 320)])
    pltpu.sync_copy(zbuf.at[pl.ds(0, 312)], acc.at[pl.ds(base + 320, 312)])
    plsc.subcore_barrier()

    def body(j, carry):
        pltpu.sync_copy(ones_v, acc.at[colv.at[j]], add=True)
        return carry

    lax.fori_loop(0, NCHUNK_A, body, 0)
    plsc.subcore_barrier()
    pltpu.sync_copy(acc.at[pl.ds(base, STRIPE)],
                    out_hbm.at[c, pl.ds(base, STRIPE)])


@functools.partial(
    pl.kernel,
    out_type=jax.ShapeDtypeStruct((2 * NC, NPAD, HQ), jnp.float32),
    mesh=_mesh,
    scratch_types=[
        pltpu.VMEM((NG, GSZ), jnp.int32),
        pltpu.VMEM((NG, GSZ), jnp.int32),
        pltpu.VMEM((NBUF, GSZ, HQ), jnp.float32),
        pltpu.VMEM((128, HQ), jnp.float32),
        pltpu.VMEM_SHARED((NPAD, HQ), jnp.float32),
        pltpu.VMEM_SHARED((NPAD, HQ), jnp.float32),
        [pltpu.SemaphoreType.DMA] * NBUF,
        [pltpu.SemaphoreType.DMA] * NBUF,
    ],
    compiler_params=pltpu.CompilerParams(use_tc_tiling_on_sc=False),
)
def _agg(row_hbm, col_hbm, y_hbm, out_hbm, rowv, colv, msg, zbuf, acc,
         ystage, gsem, ssem):
    c = lax.axis_index("c")
    s = lax.axis_index("s")
    base = s * STRIPE
    pltpu.sync_copy(row_hbm.at[s], rowv)
    pltpu.sync_copy(col_hbm.at[s], colv)

    def fz(i, carry):
        def fz2(j, carry2):
            zbuf[i, pl.ds(j * 16, 16)] = jnp.zeros((16,), jnp.float32)
            return carry2

        return lax.fori_loop(0, HQ // 16, fz2, carry)

    lax.fori_loop(0, 128, fz, 0)

    def gather(j, b):
        return pltpu.async_copy(ystage.at[rowv.at[j]], msg.at[b], gsem[b])

    for p in range(2):
        q = c * 2 + p
        pltpu.sync_copy(y_hbm.at[pl.ds(q * NPAD + base, STRIPE)],
                        ystage.at[pl.ds(base, STRIPE)])
        for k in range(4):
            pltpu.sync_copy(zbuf, acc.at[pl.ds(base + 128 * k, 128)])
        pltpu.sync_copy(zbuf.at[pl.ds(0, STRIPE - 512)],
                        acc.at[pl.ds(base + 512, STRIPE - 512)])
        plsc.subcore_barrier()

        for b in range(NBUF):
            gather(b, b)

        def outer(i, carry):
            j0 = i * NBUF
            for b in range(NBUF):
                pltpu.make_async_copy(
                    ystage.at[rowv.at[j0 + b]], msg.at[b], gsem[b]).wait()
                pltpu.async_copy(msg.at[b], acc.at[colv.at[j0 + b]],
                                 ssem[b], add=True)
            for b in range(NBUF):
                pltpu.make_async_copy(
                    msg.at[b], acc.at[colv.at[j0 + b]], ssem[b]).wait()

                @pl.when(i < NG // NBUF - 1)
                def _():
                    gather(j0 + NBUF + b, b)

            return carry

        lax.fori_loop(0, NG // NBUF, outer, 0)
        plsc.subcore_barrier()
        pltpu.sync_copy(acc.at[pl.ds(base, STRIPE)],
                        out_hbm.at[q, pl.ds(base, STRIPE)])


def _tc_pre_body(degp_ref, x_ref, w_ref, y_ref, dinv_ref):
    deg = 1.0 + degp_ref[0, :, 0:1] + degp_ref[1, :, 0:1]
    dinv = 1.0 / jnp.sqrt(deg)
    xw = jnp.dot(x_ref[...], w_ref[...], preferred_element_type=jnp.float32)
    y = dinv[:N] * xw
    for q in range(4):
        y_ref[q, :N, :] = y[:, q * HQ:(q + 1) * HQ]
    dinv_ref[...] = dinv


BR = NPAD // 8


def _tc_step_body(aggp_ref, y_ref, dinv_ref, b_ref, w_ref, out_ref):
    dinv = dinv_ref[...]
    hw = jnp.zeros((BR, 128), jnp.float32)
    for q in range(4):
        hq = jnp.maximum(
            dinv * (aggp_ref[q] + y_ref[q]) + b_ref[:, q * HQ:(q + 1) * HQ],
            0.0)
        hw = hw + jnp.dot(hq, w_ref[q * HQ:(q + 1) * HQ, :],
                          preferred_element_type=jnp.float32)
    y_next = dinv * hw
    for q in range(4):
        out_ref[q] = y_next[:, q * HQ:(q + 1) * HQ]


def _tc_fin_body(y4_ref, dinv_ref, fw1_ref, fb1_ref, fw2_ref, fb2_ref, q_ref):
    h3 = jnp.concatenate(
        [y4_ref[0, :N, :], y4_ref[1, :N, :]], axis=1) / dinv_ref[:N, :]
    g = jnp.mean(h3, axis=0, keepdims=True)
    g2 = jnp.maximum(
        jnp.dot(g, fw1_ref[...], preferred_element_type=jnp.float32)
        + fb1_ref[...], 0.0)
    q_ref[...] = (jnp.dot(g2, fw2_ref[...], preferred_element_type=jnp.float32)
                  + fb2_ref[...])


def kernel(x, edge_index, W1, b1, W2, b2, W3, b3, fw1, fb1, fw2, fb2):
    row = edge_index[0].astype(jnp.int32)
    col = edge_index[1].astype(jnp.int32)
    col_a = jnp.concatenate(
        [col, jnp.full((EPAD_A - E,), N, jnp.int32)]).reshape(
            NW, NCHUNK_A, CHUNK)
    row_b = jnp.concatenate(
        [row, jnp.zeros((EPAD_B - E,), jnp.int32)]).reshape(NS, NG, GSZ)
    pad_dst = N + jnp.arange(EPAD_B - E, dtype=jnp.int32) % (NPAD - N)
    col_b = jnp.concatenate([col, pad_dst]).reshape(NS, NG, GSZ)

    degp = _deg_kernel(col_a)

    y1, dinv = pl.pallas_call(
        _tc_pre_body,
        out_shape=[jax.ShapeDtypeStruct((4, NPAD, HQ), jnp.float32),
                   jax.ShapeDtypeStruct((NPAD, 1), jnp.float32)],
    )(degp, x, W1)

    w3p = jnp.concatenate([W3, jnp.zeros((128, 64), jnp.float32)], axis=1)
    w_st = jnp.stack([W2, w3p, jnp.eye(128, dtype=jnp.float32)])
    b3p = jnp.concatenate([b3, jnp.zeros((64,), jnp.float32)])
    b_st = jnp.stack([b1.reshape(1, 128), b2.reshape(1, 128),
                      b3p.reshape(1, 128)])

    qblock = pl.BlockSpec((4, BR, HQ), lambda i: (0, i, 0))
    step_tc = pl.pallas_call(
        _tc_step_body,
        grid=(NPAD // BR,),
        in_specs=[
            qblock,
            qblock,
            pl.BlockSpec((BR, 1), lambda i: (i, 0)),
            pl.BlockSpec((1, 128), lambda i: (0, 0)),
            pl.BlockSpec((128, 128), lambda i: (0, 0)),
        ],
        out_specs=qblock,
        out_shape=jax.ShapeDtypeStruct((4, NPAD, HQ), jnp.float32),
    )

    def step(y, wb):
        w_i, b_i = wb
        aggp = _agg(row_b, col_b, y.reshape(4 * NPAD, HQ))
        aggp = aggp.reshape(4, NPAD, HQ)
        return step_tc(aggp, y, dinv, b_i, w_i), None

    y4, _ = lax.scan(step, y1, (w_st, b_st))

    q = pl.pallas_call(
        _tc_fin_body,
        out_shape=jax.ShapeDtypeStruct((1, 64), jnp.float32),
    )(y4, dinv, fw1, fb1.reshape(1, 32), fw2, fb2.reshape(1, 64))
    return q
